# baseline (speedup 1.0000x reference)
import functools

import jax
import jax.numpy as jnp
from jax import lax
from jax.experimental import pallas as pl
from jax.experimental.pallas import tpu as pltpu

_BN_EPS = 1e-5
_VMEM_LIMIT = 64 * 1024 * 1024


def _flatten_pad(xflat, pad, h, w):
    xq = xflat.astype(jnp.bfloat16)
    x3 = xq.reshape(xflat.shape[0], h, w)
    xp = jnp.pad(x3, ((0, 0), (pad, 2 * pad), (pad, 4 * pad)))
    return xp.reshape(xflat.shape[0], -1)


def _patches(xf, taps, flat_len):
    return jnp.concatenate([xf[:, off:off + flat_len] for off in taps], axis=0)


def _stats_kernel(x_ref, w_ref, sum_ref, sq_ref, *, pad, taps, flat_len, wp,
                  w_valid, h_valid):
    xf = _flatten_pad(x_ref[0], pad, h_valid, w_valid)
    y = jnp.dot(w_ref[...], _patches(xf, taps, flat_len),
                preferred_element_type=jnp.float32)
    col = lax.broadcasted_iota(jnp.int32, y.shape, 1)
    ym = jnp.where(col % wp < w_valid, y, 0.0)
    sum_ref[0] = jnp.sum(ym, axis=1, keepdims=True)
    sq_ref[0] = jnp.sum(ym * ym, axis=1, keepdims=True)


def _conv_bn_sigmoid_kernel(x_ref, w_ref, scale_ref, shift_ref, o_ref, *, pad,
                            taps, flat_len, wp, w_valid, h_valid):
    c_out = o_ref.shape[1]
    scale = scale_ref[...]
    shift = shift_ref[...]
    xf = _flatten_pad(x_ref[0], pad, h_valid, w_valid)
    y = jnp.dot(w_ref[...], _patches(xf, taps, flat_len),
                preferred_element_type=jnp.float32)
    z = y * scale + shift
    s = pl.reciprocal(1.0 + jnp.exp(-z), approx=False)
    s = s.reshape(c_out, h_valid, wp)[:, :, :w_valid]
    o_ref[0] = s.reshape(c_out, h_valid * w_valid)


@jax.jit
def kernel(x, conv_w, conv_b, bn_gamma, bn_beta):
    del conv_b
    n, c_in, h, w = x.shape
    c_out, _, kh, kw = conv_w.shape
    pad = 1
    hp, wp = h + 3 * pad, w + 5 * pad
    flat_len = h * wp
    taps = tuple(dy * wp + dx for dy in range(kh) for dx in range(kw))
    assert hp * wp >= flat_len + taps[-1]

    wmat = conv_w.transpose(0, 2, 3, 1).reshape(c_out, kh * kw * c_in)
    wmat = wmat.astype(jnp.bfloat16)

    xflat = x.reshape(n, c_in, h * w)
    kw_common = dict(pad=pad, taps=taps, flat_len=flat_len, wp=wp, w_valid=w,
                     h_valid=h)
    psum, psq = pl.pallas_call(
        functools.partial(_stats_kernel, **kw_common),
        out_shape=(
            jax.ShapeDtypeStruct((n, c_out, 1), jnp.float32),
            jax.ShapeDtypeStruct((n, c_out, 1), jnp.float32),
        ),
        grid=(n,),
        in_specs=[
            pl.BlockSpec((1, c_in, h * w), lambda i: (i, 0, 0)),
            pl.BlockSpec((c_out, kh * kw * c_in), lambda i: (0, 0)),
        ],
        out_specs=(
            pl.BlockSpec((1, c_out, 1), lambda i: (i, 0, 0)),
            pl.BlockSpec((1, c_out, 1), lambda i: (i, 0, 0)),
        ),
        compiler_params=pltpu.CompilerParams(
            dimension_semantics=("arbitrary",),
            vmem_limit_bytes=_VMEM_LIMIT),
    )(xflat, wmat)

    inv_m = 1.0 / float(n * h * w)
    sum_y = jnp.sum(psum, axis=0)
    sum_y2 = jnp.sum(psq, axis=0)
    mean = sum_y * inv_m
    var = jnp.maximum(sum_y2 * inv_m - mean * mean, 0.0)
    scale = bn_gamma.reshape(c_out, 1) * lax.rsqrt(var + _BN_EPS)
    shift = bn_beta.reshape(c_out, 1) - mean * scale

    out = pl.pallas_call(
        functools.partial(_conv_bn_sigmoid_kernel, **kw_common),
        out_shape=jax.ShapeDtypeStruct((n, c_out, h * w), jnp.float32),
        grid=(n,),
        in_specs=[
            pl.BlockSpec((1, c_in, h * w), lambda i: (i, 0, 0)),
            pl.BlockSpec((c_out, kh * kw * c_in), lambda i: (0, 0)),
            pl.BlockSpec((c_out, 1), lambda i: (0, 0)),
            pl.BlockSpec((c_out, 1), lambda i: (0, 0)),
        ],
        out_specs=pl.BlockSpec((1, c_out, h * w), lambda i: (i, 0, 0)),
        compiler_params=pltpu.CompilerParams(
            dimension_semantics=("arbitrary",),
            vmem_limit_bytes=_VMEM_LIMIT),
    )(xflat, wmat, scale, shift)
    return out.reshape(n, c_out, h, w)


# speedup vs baseline: 1.2515x; 1.2515x over previous
import functools

import jax
import jax.numpy as jnp
from jax import lax
from jax.experimental import pallas as pl
from jax.experimental.pallas import tpu as pltpu

_BN_EPS = 1e-5
_VMEM_LIMIT = 64 * 1024 * 1024


def _flatten_pad(xflat, pad, h, w):
    xq = xflat.astype(jnp.bfloat16)
    x3 = xq.reshape(xflat.shape[0], h, w)
    xp = jnp.pad(x3, ((0, 0), (pad, 2 * pad), (pad, 4 * pad)))
    return xp.reshape(xflat.shape[0], -1)


def _patches(xf, taps, flat_len):
    return jnp.concatenate([xf[:, off:off + flat_len] for off in taps], axis=0)


def _fused_kernel(x_ref, w_ref, gamma_ref, beta_ref, o_ref, y_scr, acc_scr, *,
                  pad, taps, flat_len, wp, w_valid, h_valid, inv_m):
    phase = pl.program_id(0)
    i = pl.program_id(1)

    @pl.when(phase == 0)
    def _conv_and_stats():
        xf = _flatten_pad(x_ref[0], pad, h_valid, w_valid)
        y = jnp.dot(w_ref[...], _patches(xf, taps, flat_len),
                    preferred_element_type=jnp.float32)
        col = lax.broadcasted_iota(jnp.int32, y.shape, 1)
        ym = jnp.where(col % wp < w_valid, y, 0.0)
        y_scr[i] = ym.astype(y_scr.dtype)
        part = jnp.concatenate(
            [jnp.sum(ym, axis=1, keepdims=True),
             jnp.sum(ym * ym, axis=1, keepdims=True)], axis=1)

        @pl.when(i == 0)
        def _init():
            acc_scr[...] = part

        @pl.when(i > 0)
        def _accum():
            acc_scr[...] = acc_scr[...] + part

    @pl.when(phase == 1)
    def _bn_sigmoid():
        c_out = o_ref.shape[1]
        mean = acc_scr[:, 0:1] * inv_m
        var = jnp.maximum(acc_scr[:, 1:2] * inv_m - mean * mean, 0.0)
        scale = gamma_ref[...] * lax.rsqrt(var + _BN_EPS)
        shift = beta_ref[...] - mean * scale
        z = y_scr[i].astype(jnp.float32) * scale + shift
        s = pl.reciprocal(1.0 + jnp.exp(-z), approx=False)
        s = s.reshape(c_out, h_valid, wp)[:, :, :w_valid]
        o_ref[0] = s.reshape(c_out, h_valid * w_valid)


@jax.jit
def kernel(x, conv_w, conv_b, bn_gamma, bn_beta):
    del conv_b
    n, c_in, h, w = x.shape
    c_out, _, kh, kw = conv_w.shape
    pad = 1
    hp, wp = h + 3 * pad, w + 5 * pad
    flat_len = h * wp
    taps = tuple(dy * wp + dx for dy in range(kh) for dx in range(kw))
    assert hp * wp >= flat_len + taps[-1]

    wmat = conv_w.transpose(0, 2, 3, 1).reshape(c_out, kh * kw * c_in)
    wmat = wmat.astype(jnp.bfloat16)
    xflat = x.reshape(n, c_in, h * w)

    out = pl.pallas_call(
        functools.partial(_fused_kernel, pad=pad, taps=taps,
                          flat_len=flat_len, wp=wp, w_valid=w, h_valid=h,
                          inv_m=1.0 / float(n * h * w)),
        out_shape=jax.ShapeDtypeStruct((n, c_out, h * w), jnp.float32),
        grid=(2, n),
        in_specs=[
            pl.BlockSpec((1, c_in, h * w), lambda p, i: ((1 - p) * i, 0, 0)),
            pl.BlockSpec((c_out, kh * kw * c_in), lambda p, i: (0, 0)),
            pl.BlockSpec((c_out, 1), lambda p, i: (0, 0)),
            pl.BlockSpec((c_out, 1), lambda p, i: (0, 0)),
        ],
        out_specs=pl.BlockSpec((1, c_out, h * w), lambda p, i: (p * i, 0, 0)),
        scratch_shapes=[
            pltpu.VMEM((n, c_out, flat_len), jnp.bfloat16),
            pltpu.VMEM((c_out, 2), jnp.float32),
        ],
        compiler_params=pltpu.CompilerParams(
            dimension_semantics=("arbitrary", "arbitrary"),
            vmem_limit_bytes=_VMEM_LIMIT),
    )(xflat, wmat, bn_gamma.reshape(c_out, 1), bn_beta.reshape(c_out, 1))
    return out.reshape(n, c_out, h, w)
